# baseline (speedup 1.0000x reference)
import dataclasses
import functools

import jax
import jax.numpy as jnp
from jax import lax
from jax.experimental import pallas as pl
from jax.experimental.pallas import tpu as pltpu
from jax.experimental.pallas import tpu_sc as plsc

NUM_CLASSES = 80
MAX_BOXES = 100
N = 60
L = 16
NVEC = 4


def _f32(*shape):
    return jax.ShapeDtypeStruct(shape, jnp.float32)


_MESH = plsc.VectorSubcoreMesh(core_axis_name="c", subcore_axis_name="s")

_CP = pltpu.CompilerParams()
if "needs_layout_passes" in pltpu.CompilerParams.__dataclass_fields__:
    _CP = dataclasses.replace(_CP, needs_layout_passes=False)


@functools.partial(
    pl.kernel,
    out_type=(
        _f32(MAX_BOXES, 4), _f32(64, 64), _f32(64, 64, NUM_CLASSES),
        _f32(MAX_BOXES, 4), _f32(32, 32), _f32(32, 32, NUM_CLASSES),
        _f32(MAX_BOXES, 4), _f32(16, 16), _f32(16, 16, NUM_CLASSES),
    ),
    mesh=_MESH,
    compiler_params=_CP,
    scratch_types=[
        pltpu.VMEM((64, 16), jnp.float32),
        pltpu.VMEM((2, 64, NUM_CLASSES), jnp.float32),
        pltpu.VMEM((1, 32, NUM_CLASSES), jnp.float32),
        pltpu.VMEM((1, 16, NUM_CLASSES), jnp.float32),
        pltpu.VMEM((2, 64), jnp.float32),
        pltpu.VMEM((1, 32), jnp.float32),
        pltpu.VMEM((1, 16), jnp.float32),
        pltpu.VMEM((MAX_BOXES, 4), jnp.float32),
    ],
)
def _sc_label_kernel(label_hbm,
                     box8_hbm, obj8_hbm, cls8_hbm,
                     box16_hbm, obj16_hbm, cls16_hbm,
                     box32_hbm, obj32_hbm, cls32_hbm,
                     lab_v, c8_v, c16_v, c32_v, o8_v, o16_v, o32_v, box_v):
    w = lax.axis_index("s") * 2 + lax.axis_index("c")
    x0 = 2 * w

    iota = lax.iota(jnp.int32, L)
    zeros16 = jnp.zeros((L,), jnp.float32)
    ones16 = jnp.ones((L,), jnp.float32)
    z16i = jnp.zeros((L,), jnp.int32)

    pltpu.sync_copy(label_hbm, lab_v)

    @pl.loop(0, 2)
    def _(i):
        @pl.loop(0, 64)
        def _(j):
            @pl.loop(0, NUM_CLASSES // L)
            def _(k):
                c8_v[i, j, pl.ds(k * L, L)] = zeros16

        @pl.loop(0, 64 // L)
        def _(k):
            o8_v[i, pl.ds(k * L, L)] = zeros16

    @pl.loop(0, 32)
    def _(j):
        @pl.loop(0, NUM_CLASSES // L)
        def _(k):
            c16_v[0, j, pl.ds(k * L, L)] = zeros16

    @pl.loop(0, 32 // L)
    def _(k):
        o16_v[0, pl.ds(k * L, L)] = zeros16

    @pl.when(w < 16)
    def _():
        @pl.loop(0, 16)
        def _(j):
            @pl.loop(0, NUM_CLASSES // L)
            def _(k):
                c32_v[0, j, pl.ds(k * L, L)] = zeros16

        o32_v[0, pl.ds(0, L)] = zeros16

    cols = []
    for v in range(NVEC):
        rows = jnp.minimum(iota + (v * L), N - 1)

        def _col(c, rows=rows):
            return plsc.load_gather(lab_v, [rows, jnp.full((L,), c, jnp.int32)])

        cls_f, xf, yf, wf, hf = _col(0), _col(1), _col(2), _col(3), _col(4)
        c8x = (xf * 0.125).astype(jnp.int32)
        c8y = (yf * 0.125).astype(jnp.int32)
        c16x = lax.shift_right_logical(c8x, 4)
        c16y = lax.shift_right_logical(c8y, 4)
        c32x = lax.shift_right_logical(c16x, 5)
        c32y = lax.shift_right_logical(c16y, 5)
        cls_i = cls_f.astype(jnp.int32)
        valid = iota < (N - v * L)
        cols.append(dict(
            rows=rows, valid=valid, xf=xf, yf=yf, wf=wf, hf=hf,
            cls_i=cls_i, c8x=c8x, c8y=c8y, c16x=c16x, c16y=c16y,
            c32x=c32x, c32y=c32y,
            c8xf=c8x.astype(jnp.float32), c8yf=c8y.astype(jnp.float32),
            c16xf=c16x.astype(jnp.float32), c16yf=c16y.astype(jnp.float32),
        ))

    for v in range(NVEC):
        cv = cols[v]
        in8 = cv["valid"] & (cv["c8x"] >= x0) & (cv["c8x"] < x0 + 2)
        lx8 = jnp.minimum(jnp.maximum(cv["c8x"] - x0, 0), 1)
        in16 = cv["valid"] & (cv["c16x"] == w)
        in32 = cv["valid"] & (cv["c32x"] == w)

        @pl.loop(0, L)
        def _(l, cv=cv, in8=in8, lx8=lx8, in16=in16, in32=in32):
            lm = iota == l
            plsc.addupdate_scatter(c8_v, [lx8, cv["c8y"], cv["cls_i"]],
                                   ones16, mask=lm & in8)
            plsc.addupdate_scatter(o8_v, [lx8, cv["c8y"]],
                                   ones16, mask=lm & in8)
            plsc.addupdate_scatter(c16_v, [z16i, cv["c16y"], cv["cls_i"]],
                                   ones16, mask=lm & in16)
            plsc.addupdate_scatter(o16_v, [z16i, cv["c16y"]],
                                   ones16, mask=lm & in16)
            plsc.addupdate_scatter(c32_v, [z16i, cv["c32y"], cv["cls_i"]],
                                   ones16, mask=lm & in32)
            plsc.addupdate_scatter(o32_v, [z16i, cv["c32y"]],
                                   ones16, mask=lm & in32)

    def _build_box(dst_hbm, k0, k1):
        @pl.loop(0, MAX_BOXES * 4 // L)
        def _(k):
            flat = iota + k * L
            plsc.store_scatter(
                box_v, [lax.shift_right_logical(flat, 2), flat & 3], zeros16)

        for v in range(NVEC):
            cv = cols[v]
            for c, val in ((0, cv[k0]), (1, cv[k1]),
                           (2, cv["wf"]), (3, cv["hf"])):
                plsc.store_scatter(box_v,
                                   [cv["rows"], jnp.full((L,), c, jnp.int32)],
                                   val, mask=cv["valid"])
        pltpu.sync_copy(box_v, dst_hbm)

    @pl.when(w == 16)
    def _():
        _build_box(box8_hbm, "xf", "yf")

    @pl.when(w == 17)
    def _():
        _build_box(box16_hbm, "c8xf", "c8yf")

    @pl.when(w == 18)
    def _():
        _build_box(box32_hbm, "c16xf", "c16yf")

    pltpu.sync_copy(c8_v, cls8_hbm.at[pl.ds(x0, 2)])
    pltpu.sync_copy(o8_v, obj8_hbm.at[pl.ds(x0, 2)])
    pltpu.sync_copy(c16_v, cls16_hbm.at[pl.ds(w, 1)])
    pltpu.sync_copy(o16_v, obj16_hbm.at[pl.ds(w, 1)])

    @pl.when(w < 16)
    def _():
        pltpu.sync_copy(c32_v, cls32_hbm.at[pl.ds(w, 1)])
        pltpu.sync_copy(o32_v, obj32_hbm.at[pl.ds(w, 1)])


def kernel(image, label):
    lab = jnp.zeros((64, 16), jnp.float32).at[:N, :5].set(label)
    (box8, obj8, cls8, box16, obj16, cls16,
     box32, obj32, cls32) = _sc_label_kernel(lab)
    return (image, box8, obj8, cls8, box16, obj16, cls16,
            box32, obj32, cls32)


# speedup vs baseline: 1.0024x; 1.0024x over previous
import dataclasses
import functools

import jax
import jax.numpy as jnp
from jax import lax
from jax.experimental import pallas as pl
from jax.experimental.pallas import tpu as pltpu
from jax.experimental.pallas import tpu_sc as plsc

NUM_CLASSES = 80
MAX_BOXES = 100
N = 60
L = 16
NVEC = 4


def _f32(*shape):
    return jax.ShapeDtypeStruct(shape, jnp.float32)


_MESH = plsc.VectorSubcoreMesh(core_axis_name="c", subcore_axis_name="s")

_CP = pltpu.CompilerParams()
if "needs_layout_passes" in pltpu.CompilerParams.__dataclass_fields__:
    _CP = dataclasses.replace(_CP, needs_layout_passes=False)
if "use_tc_tiling_on_sc" in pltpu.CompilerParams.__dataclass_fields__:
    _CP = dataclasses.replace(_CP, use_tc_tiling_on_sc=True)


@functools.partial(
    pl.kernel,
    out_type=(
        _f32(MAX_BOXES, 4), _f32(64, 64), _f32(64, 64, NUM_CLASSES),
        _f32(MAX_BOXES, 4), _f32(32, 32), _f32(32, 32, NUM_CLASSES),
        _f32(MAX_BOXES, 4), _f32(16, 16), _f32(16, 16, NUM_CLASSES),
    ),
    mesh=_MESH,
    compiler_params=_CP,
    scratch_types=[
        pltpu.VMEM((64, 16), jnp.float32),
        pltpu.VMEM((2, 64, NUM_CLASSES), jnp.float32),
        pltpu.VMEM((1, 32, NUM_CLASSES), jnp.float32),
        pltpu.VMEM((1, 16, NUM_CLASSES), jnp.float32),
        pltpu.VMEM((2, 64), jnp.float32),
        pltpu.VMEM((1, 32), jnp.float32),
        pltpu.VMEM((1, 16), jnp.float32),
        pltpu.VMEM((MAX_BOXES, 4), jnp.float32),
    ],
)
def _sc_label_kernel(label_hbm,
                     box8_hbm, obj8_hbm, cls8_hbm,
                     box16_hbm, obj16_hbm, cls16_hbm,
                     box32_hbm, obj32_hbm, cls32_hbm,
                     lab_v, c8_v, c16_v, c32_v, o8_v, o16_v, o32_v, box_v):
    w = lax.axis_index("s") * 2 + lax.axis_index("c")
    x0 = 2 * w

    iota = lax.iota(jnp.int32, L)
    zeros16 = jnp.zeros((L,), jnp.float32)
    ones16 = jnp.ones((L,), jnp.float32)
    z16i = jnp.zeros((L,), jnp.int32)

    pltpu.sync_copy(label_hbm, lab_v)

    @pl.loop(0, 2)
    def _(i):
        @pl.loop(0, 64)
        def _(j):
            @pl.loop(0, NUM_CLASSES // L)
            def _(k):
                c8_v[i, j, pl.ds(k * L, L)] = zeros16

        @pl.loop(0, 64 // L)
        def _(k):
            o8_v[i, pl.ds(k * L, L)] = zeros16

    @pl.loop(0, 32)
    def _(j):
        @pl.loop(0, NUM_CLASSES // L)
        def _(k):
            c16_v[0, j, pl.ds(k * L, L)] = zeros16

    @pl.loop(0, 32 // L)
    def _(k):
        o16_v[0, pl.ds(k * L, L)] = zeros16

    @pl.when(w < 16)
    def _():
        @pl.loop(0, 16)
        def _(j):
            @pl.loop(0, NUM_CLASSES // L)
            def _(k):
                c32_v[0, j, pl.ds(k * L, L)] = zeros16

        o32_v[0, pl.ds(0, L)] = zeros16

    cols = []
    for v in range(NVEC):
        rows = jnp.minimum(iota + (v * L), N - 1)

        def _col(c, rows=rows):
            return plsc.load_gather(lab_v, [rows, jnp.full((L,), c, jnp.int32)])

        cls_f, xf, yf, wf, hf = _col(0), _col(1), _col(2), _col(3), _col(4)
        c8x = (xf * 0.125).astype(jnp.int32)
        c8y = (yf * 0.125).astype(jnp.int32)
        c16x = lax.shift_right_logical(c8x, 4)
        c16y = lax.shift_right_logical(c8y, 4)
        c32x = lax.shift_right_logical(c16x, 5)
        c32y = lax.shift_right_logical(c16y, 5)
        cls_i = cls_f.astype(jnp.int32)
        valid = iota < (N - v * L)
        cols.append(dict(
            rows=rows, valid=valid, xf=xf, yf=yf, wf=wf, hf=hf,
            cls_i=cls_i, c8x=c8x, c8y=c8y, c16x=c16x, c16y=c16y,
            c32x=c32x, c32y=c32y,
            c8xf=c8x.astype(jnp.float32), c8yf=c8y.astype(jnp.float32),
            c16xf=c16x.astype(jnp.float32), c16yf=c16y.astype(jnp.float32),
        ))

    for v in range(NVEC):
        cv = cols[v]
        in8 = cv["valid"] & (cv["c8x"] >= x0) & (cv["c8x"] < x0 + 2)
        lx8 = jnp.minimum(jnp.maximum(cv["c8x"] - x0, 0), 1)
        in16 = cv["valid"] & (cv["c16x"] == w)
        in32 = cv["valid"] & (cv["c32x"] == w)

        @pl.loop(0, L)
        def _(l, cv=cv, in8=in8, lx8=lx8, in16=in16, in32=in32):
            lm = iota == l
            plsc.addupdate_scatter(c8_v, [lx8, cv["c8y"], cv["cls_i"]],
                                   ones16, mask=lm & in8)
            plsc.addupdate_scatter(o8_v, [lx8, cv["c8y"]],
                                   ones16, mask=lm & in8)
            plsc.addupdate_scatter(c16_v, [z16i, cv["c16y"], cv["cls_i"]],
                                   ones16, mask=lm & in16)
            plsc.addupdate_scatter(o16_v, [z16i, cv["c16y"]],
                                   ones16, mask=lm & in16)
            plsc.addupdate_scatter(c32_v, [z16i, cv["c32y"], cv["cls_i"]],
                                   ones16, mask=lm & in32)
            plsc.addupdate_scatter(o32_v, [z16i, cv["c32y"]],
                                   ones16, mask=lm & in32)

    def _build_box(dst_hbm, k0, k1):
        @pl.loop(0, MAX_BOXES * 4 // L)
        def _(k):
            flat = iota + k * L
            plsc.store_scatter(
                box_v, [lax.shift_right_logical(flat, 2), flat & 3], zeros16)

        for v in range(NVEC):
            cv = cols[v]
            for c, val in ((0, cv[k0]), (1, cv[k1]),
                           (2, cv["wf"]), (3, cv["hf"])):
                plsc.store_scatter(box_v,
                                   [cv["rows"], jnp.full((L,), c, jnp.int32)],
                                   val, mask=cv["valid"])
        pltpu.sync_copy(box_v, dst_hbm)

    @pl.when(w == 16)
    def _():
        _build_box(box8_hbm, "xf", "yf")

    @pl.when(w == 17)
    def _():
        _build_box(box16_hbm, "c8xf", "c8yf")

    @pl.when(w == 18)
    def _():
        _build_box(box32_hbm, "c16xf", "c16yf")

    pltpu.sync_copy(c8_v, cls8_hbm.at[pl.ds(x0, 2)])
    pltpu.sync_copy(o8_v, obj8_hbm.at[pl.ds(x0, 2)])
    pltpu.sync_copy(c16_v, cls16_hbm.at[pl.ds(w, 1)])
    pltpu.sync_copy(o16_v, obj16_hbm.at[pl.ds(w, 1)])

    @pl.when(w < 16)
    def _():
        pltpu.sync_copy(c32_v, cls32_hbm.at[pl.ds(w, 1)])
        pltpu.sync_copy(o32_v, obj32_hbm.at[pl.ds(w, 1)])


def kernel(image, label):
    lab = jnp.zeros((64, 16), jnp.float32).at[:N, :5].set(label)
    (box8, obj8, cls8, box16, obj16, cls16,
     box32, obj32, cls32) = _sc_label_kernel(lab)
    return (image, box8, obj8, cls8, box16, obj16, cls16,
            box32, obj32, cls32)


# speedup vs baseline: 1.0987x; 1.0961x over previous
import dataclasses
import functools

import jax
import jax.numpy as jnp
from jax import lax
from jax.experimental import pallas as pl
from jax.experimental.pallas import tpu as pltpu
from jax.experimental.pallas import tpu_sc as plsc

NUM_CLASSES = 80
MAX_BOXES = 100
N = 60
L = 16
NVEC = 4


def _f32(*shape):
    return jax.ShapeDtypeStruct(shape, jnp.float32)


_MESH = plsc.VectorSubcoreMesh(core_axis_name="c", subcore_axis_name="s")

_CP = pltpu.CompilerParams()
if "needs_layout_passes" in pltpu.CompilerParams.__dataclass_fields__:
    _CP = dataclasses.replace(_CP, needs_layout_passes=False)
if "use_tc_tiling_on_sc" in pltpu.CompilerParams.__dataclass_fields__:
    _CP = dataclasses.replace(_CP, use_tc_tiling_on_sc=True)


@functools.partial(
    pl.kernel,
    out_type=(
        _f32(4, 64),
        _f32(64, 64), _f32(64, 64, NUM_CLASSES),
        _f32(32, 32), _f32(32, 32, NUM_CLASSES),
        _f32(16, 16), _f32(16, 16, NUM_CLASSES),
    ),
    mesh=_MESH,
    compiler_params=_CP,
    scratch_types=[
        pltpu.VMEM((N, 5), jnp.float32),
        pltpu.VMEM((2, 64, NUM_CLASSES), jnp.float32),
        pltpu.VMEM((1, 32, NUM_CLASSES), jnp.float32),
        pltpu.VMEM((1, 16, NUM_CLASSES), jnp.float32),
        pltpu.VMEM((2, 64), jnp.float32),
        pltpu.VMEM((1, 32), jnp.float32),
        pltpu.VMEM((1, 16), jnp.float32),
        pltpu.VMEM((4, 64), jnp.float32),
        pltpu.SemaphoreType.DMA,
    ],
)
def _sc_label_kernel(label_hbm,
                     stage_hbm, obj8_hbm, cls8_hbm,
                     obj16_hbm, cls16_hbm, obj32_hbm, cls32_hbm,
                     lab_v, c8_v, c16_v, c32_v, o8_v, o16_v, o32_v,
                     stage_v, sem):
    w = lax.axis_index("s") * 2 + lax.axis_index("c")
    x0 = 2 * w

    iota = lax.iota(jnp.int32, L)
    zeros16 = jnp.zeros((L,), jnp.float32)
    ones16 = jnp.ones((L,), jnp.float32)
    z16i = jnp.zeros((L,), jnp.int32)

    pltpu.sync_copy(label_hbm, lab_v)

    cols = []
    for v in range(NVEC):
        rows = jnp.minimum(iota + (v * L), N - 1)

        def _col(c, rows=rows):
            return plsc.load_gather(lab_v, [rows, jnp.full((L,), c, jnp.int32)])

        cls_f, xf, yf = _col(0), _col(1), _col(2)
        c8x = (xf * 0.125).astype(jnp.int32)
        c8y = (yf * 0.125).astype(jnp.int32)
        c16x = lax.shift_right_logical(c8x, 4)
        c16y = lax.shift_right_logical(c8y, 4)
        c32x = lax.shift_right_logical(c16x, 5)
        c32y = lax.shift_right_logical(c16y, 5)
        valid = iota < (N - v * L)
        cols.append(dict(
            valid=valid, cls_i=cls_f.astype(jnp.int32),
            c8x=c8x, c8y=c8y, c16x=c16x, c16y=c16y, c32x=c32x, c32y=c32y,
        ))

    def _any(bools):
        r = None
        for b in bools:
            s = jnp.any(b)
            r = s if r is None else (r | s)
        return r

    handles = []

    @pl.loop(0, 2)
    def _(i):
        @pl.loop(0, 64)
        def _(j):
            for k in range(NUM_CLASSES // L):
                c8_v[i, j, pl.ds(k * L, L)] = zeros16

    for i in range(2):
        for k in range(64 // L):
            o8_v[i, pl.ds(k * L, L)] = zeros16

    in8s = [cv["valid"] & (cv["c8x"] >= x0) & (cv["c8x"] < x0 + 2)
            for cv in cols]

    @pl.when(_any(in8s))
    def _():
        for v in range(NVEC):
            cv = cols[v]
            lx8 = jnp.minimum(jnp.maximum(cv["c8x"] - x0, 0), 1)

            @pl.loop(0, L)
            def _(l, cv=cv, lx8=lx8, in8=in8s[v]):
                m = (iota == l) & in8
                plsc.addupdate_scatter(c8_v, [lx8, cv["c8y"], cv["cls_i"]],
                                       ones16, mask=m)
                plsc.addupdate_scatter(o8_v, [lx8, cv["c8y"]], ones16, mask=m)

    handles.append(pltpu.async_copy(c8_v, cls8_hbm.at[pl.ds(x0, 2)], sem))
    handles.append(pltpu.async_copy(o8_v, obj8_hbm.at[pl.ds(x0, 2)], sem))

    @pl.loop(0, 32)
    def _(j):
        for k in range(NUM_CLASSES // L):
            c16_v[0, j, pl.ds(k * L, L)] = zeros16

    for k in range(32 // L):
        o16_v[0, pl.ds(k * L, L)] = zeros16

    in16s = [cv["valid"] & (cv["c16x"] == w) for cv in cols]

    @pl.when(_any(in16s))
    def _():
        for v in range(NVEC):
            cv = cols[v]

            @pl.loop(0, L)
            def _(l, cv=cv, in16=in16s[v]):
                m = (iota == l) & in16
                plsc.addupdate_scatter(c16_v, [z16i, cv["c16y"], cv["cls_i"]],
                                       ones16, mask=m)
                plsc.addupdate_scatter(o16_v, [z16i, cv["c16y"]], ones16, mask=m)

    handles.append(pltpu.async_copy(c16_v, cls16_hbm.at[pl.ds(w, 1)], sem))
    handles.append(pltpu.async_copy(o16_v, obj16_hbm.at[pl.ds(w, 1)], sem))

    @pl.when(w < 16)
    def _():
        @pl.loop(0, 16)
        def _(j):
            for k in range(NUM_CLASSES // L):
                c32_v[0, j, pl.ds(k * L, L)] = zeros16

        o32_v[0, pl.ds(0, L)] = zeros16

        in32s = [cv["valid"] & (cv["c32x"] == w) for cv in cols]

        @pl.when(_any(in32s))
        def _():
            for v in range(NVEC):
                cv = cols[v]

                @pl.loop(0, L)
                def _(l, cv=cv, in32=in32s[v]):
                    m = (iota == l) & in32
                    plsc.addupdate_scatter(c32_v,
                                           [z16i, cv["c32y"], cv["cls_i"]],
                                           ones16, mask=m)
                    plsc.addupdate_scatter(o32_v, [z16i, cv["c32y"]],
                                           ones16, mask=m)

        pltpu.sync_copy(c32_v, cls32_hbm.at[pl.ds(w, 1)])
        pltpu.sync_copy(o32_v, obj32_hbm.at[pl.ds(w, 1)])

    @pl.when(w == 16)
    def _():
        for v in range(NVEC):
            cv = cols[v]
            stage_v[0, pl.ds(v * L, L)] = cv["c8x"].astype(jnp.float32)
            stage_v[1, pl.ds(v * L, L)] = cv["c8y"].astype(jnp.float32)
            stage_v[2, pl.ds(v * L, L)] = cv["c16x"].astype(jnp.float32)
            stage_v[3, pl.ds(v * L, L)] = cv["c16y"].astype(jnp.float32)
        pltpu.sync_copy(stage_v, stage_hbm)

    for h in handles:
        h.wait()


def kernel(image, label):
    (stage, obj8, cls8, obj16, cls16,
     obj32, cls32) = _sc_label_kernel(label)
    pad = jnp.zeros((MAX_BOXES - N, 4), jnp.float32)
    wh = label[:, 3:5]
    box8 = jnp.concatenate([label[:, 1:5], pad], axis=0)
    box16 = jnp.concatenate(
        [jnp.concatenate([stage[0:2, :N].T, wh], axis=1), pad], axis=0)
    box32 = jnp.concatenate(
        [jnp.concatenate([stage[2:4, :N].T, wh], axis=1), pad], axis=0)
    return (image, box8, obj8, cls8, box16, obj16, cls16,
            box32, obj32, cls32)
